# Initial kernel scaffold; baseline (speedup 1.0000x reference)
import jax
import jax.numpy as jnp
from jax.experimental import pallas as pl


def kernel(edge_index, node_atts, emb, msg_w_f, msg_b_f, gru_wih_f, gru_whh_f, gru_bih_f, gru_bhh_f, msg_w_b, msg_b_b, gru_wih_b, gru_whh_b, gru_bih_b, gru_bhh_b):
    raise NotImplementedError("write your pallas kernel here")


# speedup vs baseline: 4.8061x; 4.8061x over previous
import functools

import jax
import jax.numpy as jnp
from jax import lax
from jax.experimental import pallas as pl
from jax.experimental.pallas import tpu as pltpu
from jax.experimental.pallas import tpu_sc as plsc

N = 10000
HID = 64
NDIM = 128
LAYERS = 3
E = 320000
ATTS = 16

NC = 2
NS = 16
CHUNK = 128
NP = 10112
RPT = NP // NS
G = -(-E // (NS * CHUNK))
EP = G * CHUNK * NS


def _sc_mesh():
    return plsc.VectorSubcoreMesh(
        core_axis_name="c", subcore_axis_name="s", num_cores=NC,
        num_subcores=NS)


def _rd(x):
    return x.astype(jnp.bfloat16).astype(jnp.float32)


def _scatter_body(h2cat, gidx, sidx, zed, out, gi_v, si_v, rows, acc, sem):
    c = lax.axis_index("c")
    s = lax.axis_index("s")
    pltpu.sync_copy(zed.at[pl.ds(s * RPT, RPT)], acc.at[pl.ds(s * RPT, RPT)])
    pltpu.sync_copy(gidx.at[c, s], gi_v)
    pltpu.sync_copy(sidx.at[c, s], si_v)
    plsc.subcore_barrier()

    def body(g, carry):
        pltpu.async_copy(h2cat.at[gi_v.at[g]], rows, sem).wait()
        pltpu.sync_copy(rows, acc.at[si_v.at[g]], add=True)
        return carry

    lax.fori_loop(0, G, body, 0)
    plsc.subcore_barrier()
    pltpu.sync_copy(acc.at[pl.ds(s * RPT, RPT)],
                    out.at[c, pl.ds(s * RPT, RPT)])


_SCATTER = pl.kernel(
    _scatter_body,
    out_type=jax.ShapeDtypeStruct((NC, NP, HID), jnp.float32),
    mesh=_sc_mesh(),
    scratch_types=[
        pltpu.VMEM((G, CHUNK), jnp.int32),
        pltpu.VMEM((G, CHUNK), jnp.int32),
        pltpu.VMEM((CHUNK, HID), jnp.float32),
        pltpu.VMEM_SHARED((NP, HID), jnp.float32),
        pltpu.SemaphoreType.DMA,
    ],
    compiler_params=pltpu.CompilerParams(use_tc_tiling_on_sc=False),
)


def _msg_body(h_ref, wma_ref, out_ref):
    out_ref[0] = jnp.dot(_rd(h_ref[0]), _rd(wma_ref[0]),
                         preferred_element_type=jnp.float32,
                         precision=lax.Precision.HIGHEST)


def _msg_call(h2, wma2):
    return pl.pallas_call(
        _msg_body,
        grid=(NC,),
        in_specs=[
            pl.BlockSpec((1, NP, HID), lambda d: (d, 0, 0)),
            pl.BlockSpec((1, HID, NDIM), lambda d: (d, 0, 0)),
        ],
        out_specs=pl.BlockSpec((1, NP, NDIM), lambda d: (d, 0, 0)),
        out_shape=jax.ShapeDtypeStruct((NC, NP, NDIM), jnp.float32),
    )(h2, wma2)


def _deg_body(sidx, ones_h, zed16, out, si_v, ones_v, acc):
    c = lax.axis_index("c")
    s = lax.axis_index("s")
    pltpu.sync_copy(zed16.at[pl.ds(s * RPT, RPT)], acc.at[pl.ds(s * RPT, RPT)])
    pltpu.sync_copy(sidx.at[c, s], si_v)
    pltpu.sync_copy(ones_h, ones_v)
    plsc.subcore_barrier()

    def body(g, carry):
        pltpu.sync_copy(ones_v, acc.at[si_v.at[g]], add=True)
        return carry

    lax.fori_loop(0, G, body, 0)
    plsc.subcore_barrier()
    pltpu.sync_copy(acc.at[pl.ds(s * RPT, RPT)],
                    out.at[c, pl.ds(s * RPT, RPT)])


_DEG = pl.kernel(
    _deg_body,
    out_type=jax.ShapeDtypeStruct((NC, NP, 16), jnp.float32),
    mesh=_sc_mesh(),
    scratch_types=[
        pltpu.VMEM((G, CHUNK), jnp.int32),
        pltpu.VMEM((CHUNK, 16), jnp.float32),
        pltpu.VMEM_SHARED((NP, 16), jnp.float32),
    ],
    compiler_params=pltpu.CompilerParams(use_tc_tiling_on_sc=False),
)


def _h0_body(atts_ref, emb_ref, out_ref):
    atts = atts_ref[0]
    emb = emb_ref[...]
    acc = jnp.zeros((NP, HID), jnp.float32)
    for a in range(ATTS):
        acc = acc + jnp.where(atts == a, 1.0, 0.0) * emb[a:a + 1]
    out_ref[0] = acc


def _h0_call(atts2, emb):
    return pl.pallas_call(
        _h0_body,
        grid=(NC,),
        in_specs=[
            pl.BlockSpec((1, NP, 1), lambda d: (d, 0, 0)),
            pl.BlockSpec((ATTS, HID), lambda d: (0, 0)),
        ],
        out_specs=pl.BlockSpec((1, NP, HID), lambda d: (d, 0, 0)),
        out_shape=jax.ShapeDtypeStruct((NC, NP, HID), jnp.float32),
    )(atts2, emb)


BR = NP // 8


def _dense_body(h_ref, s_ref, deg_ref, wmb_ref, bm_ref, wih_ref,
                whh_ref, bias_ref, out_ref):
    h = h_ref[0]
    S = s_ref[0]
    deg = deg_ref[0]
    wmb = wmb_ref[0]
    bm = bm_ref[0, 0:1]
    wih = wih_ref[0]
    whh = whh_ref[0]
    bias = bias_ref[0]

    dot = functools.partial(jnp.dot, preferred_element_type=jnp.float32,
                            precision=lax.Precision.HIGHEST)
    v = dot(_rd(h), _rd(wmb))
    aggr = S + deg * (v + bm)

    aggr_r = _rd(aggr)
    hr = _rd(h)
    gi_r = dot(aggr_r, wih[0])
    gi_z = dot(aggr_r, wih[1])
    gi_n = dot(aggr_r, wih[2])
    gh_r = dot(hr, whh[0])
    gh_z = dot(hr, whh[1])
    gh_n = dot(hr, whh[2])

    r = jax.nn.sigmoid(gi_r + gh_r + bias[0:1])
    z = jax.nn.sigmoid(gi_z + gh_z + bias[1:2])
    n = jnp.tanh(gi_n + bias[2:3] + r * (gh_n + bias[3:4]))
    out_ref[0] = (1.0 - z) * n + z * h


def _dense_call(h2, S2, deg2, wmb2, bm2, wih2, whh2, bias2):
    return pl.pallas_call(
        _dense_body,
        grid=(NC, NP // BR),
        in_specs=[
            pl.BlockSpec((1, BR, HID), lambda d, i: (d, i, 0)),
            pl.BlockSpec((1, BR, NDIM), lambda d, i: (d, i, 0)),
            pl.BlockSpec((1, BR, 1), lambda d, i: (d, i, 0)),
            pl.BlockSpec((1, HID, NDIM), lambda d, i: (d, 0, 0)),
            pl.BlockSpec((1, 8, NDIM), lambda d, i: (d, 0, 0)),
            pl.BlockSpec((1, 3, NDIM, HID), lambda d, i: (d, 0, 0, 0)),
            pl.BlockSpec((1, 3, HID, HID), lambda d, i: (d, 0, 0, 0)),
            pl.BlockSpec((1, 8, HID), lambda d, i: (d, 0, 0)),
        ],
        out_specs=pl.BlockSpec((1, BR, HID), lambda d, i: (d, i, 0)),
        out_shape=jax.ShapeDtypeStruct((NC, NP, HID), jnp.float32),
    )(h2, S2, deg2, wmb2, bm2, wih2, whh2, bias2)


def _prep_layer(l, msg_w_f, msg_b_f, wih_f, whh_f, bih_f, bhh_f,
                msg_w_b, msg_b_b, wih_b, whh_b, bih_b, bhh_b):
    def one(msg_w, msg_b, wih, whh, bih, bhh):
        wma = msg_w[l][:, :HID].T
        wmb = msg_w[l][:, HID:].T
        bm = jnp.zeros((8, NDIM), jnp.float32).at[0].set(msg_b[l])
        wih3 = _rd(wih[l].reshape(3, HID, NDIM).transpose(0, 2, 1))
        whh3 = _rd(whh[l].reshape(3, HID, HID).transpose(0, 2, 1))
        bi = bih[l].reshape(3, HID)
        bh = bhh[l].reshape(3, HID)
        bias = jnp.zeros((8, HID), jnp.float32)
        bias = bias.at[0].set(bi[0] + bh[0])
        bias = bias.at[1].set(bi[1] + bh[1])
        bias = bias.at[2].set(bi[2])
        bias = bias.at[3].set(bh[2])
        return wma, wmb, bm, wih3, whh3, bias

    f = one(msg_w_f, msg_b_f, wih_f, whh_f, bih_f, bhh_f)
    b = one(msg_w_b, msg_b_b, wih_b, whh_b, bih_b, bhh_b)
    return tuple(jnp.stack([x, y]) for x, y in zip(f, b))


def kernel(edge_index, node_atts, emb, msg_w_f, msg_b_f, gru_wih_f,
           gru_whh_f, gru_bih_f, gru_bhh_f, msg_w_b, msg_b_b, gru_wih_b,
           gru_whh_b, gru_bih_b, gru_bhh_b):
    src = edge_index[0]
    dst = edge_index[1]
    pad = jnp.full((EP - E,), N, jnp.int32)
    srcp = jnp.concatenate([src, pad]).reshape(NS, G, CHUNK)
    dstp = jnp.concatenate([dst, pad]).reshape(NS, G, CHUNK)
    gidx = jnp.stack([srcp, dstp + NP])
    sidx = jnp.stack([dstp, srcp])

    atts_p = jnp.pad(node_atts, (0, NP - N)).reshape(1, NP, 1)
    atts2 = jnp.broadcast_to(atts_p, (NC, NP, 1))
    zed = jnp.zeros((NP, HID), jnp.float32)
    zed16 = jnp.zeros((NP, 16), jnp.float32)
    ones16 = jnp.ones((CHUNK, 16), jnp.float32)

    h2 = _h0_call(atts2, emb)
    deg2 = _DEG(sidx, ones16, zed16)[:, :, 0:1]

    for l in range(LAYERS):
        wma2, wmb2, bm2, wih2, whh2, bias2 = _prep_layer(
            l, msg_w_f, msg_b_f, gru_wih_f, gru_whh_f, gru_bih_f, gru_bhh_f,
            msg_w_b, msg_b_b, gru_wih_b, gru_whh_b, gru_bih_b, gru_bhh_b)
        u2 = _msg_call(h2, wma2)
        SL = _SCATTER(u2[:, :, :HID].reshape(NC * NP, HID), gidx, sidx, zed)
        SR = _SCATTER(u2[:, :, HID:].reshape(NC * NP, HID), gidx, sidx, zed)
        S2 = jnp.concatenate([SL, SR], axis=2)
        h2 = _dense_call(h2, S2, deg2, wmb2, bm2, wih2, whh2, bias2)

    return jnp.concatenate([h2[0, :N], h2[1, :N]], axis=1)
